# baseline (speedup 1.0000x reference)
import functools

import jax
import jax.numpy as jnp
from jax import lax
from jax.experimental import pallas as pl
from jax.experimental.pallas import tpu as pltpu
from jax.experimental.pallas import tpu_sc as plsc

_W = 128
_NC, _NS = 2, 16


@functools.lru_cache(maxsize=None)
def _build(N, D, C):
    nworkers = _NC * _NS
    R = N // nworkers
    nw = R // _W
    mesh = plsc.VectorSubcoreMesh(core_axis_name="c", subcore_axis_name="s")

    @functools.partial(
        pl.kernel,
        out_type=jax.ShapeDtypeStruct((N, D), jnp.float32),
        mesh=mesh,
        scratch_types=[
            pltpu.VMEM((R,), jnp.int32),
            pltpu.VMEM((R,), jnp.int32),
            pltpu.VMEM((_W, D), jnp.float32),
            pltpu.VMEM((_W, D), jnp.float32),
            pltpu.VMEM((_W, D), jnp.float32),
            pltpu.VMEM((_W, D), jnp.float32),
            pltpu.VMEM_SHARED((C, D), jnp.float32),
            pltpu.SemaphoreType.DMA,
            pltpu.SemaphoreType.DMA,
            pltpu.SemaphoreType.DMA,
            pltpu.SemaphoreType.DMA,
            pltpu.SemaphoreType.DMA,
            pltpu.SemaphoreType.DMA,
        ],
    )
    def k(seq_hbm, cidx_hbm, tok_hbm, comb_hbm, out_hbm,
          i_all, ci_all, o0, o1, a0, a1, comb_sh,
          st0, st1, sc0, sc1, so0, so1):
        sid = lax.axis_index("s")
        wid = sid * _NC + lax.axis_index("c")
        base = wid * R

        @pl.when(sid == 0)
        def _():
            pltpu.sync_copy(comb_hbm, comb_sh)

        plsc.subcore_barrier()
        pltpu.sync_copy(seq_hbm.at[pl.ds(base, R)], i_all)
        pltpu.sync_copy(cidx_hbm.at[pl.ds(base, R)], ci_all)

        def issue(g, o_v, a_v, st, sc_):
            pltpu.async_copy(tok_hbm.at[i_all.at[pl.ds(g * _W, _W)]], o_v, st)
            pltpu.async_copy(comb_sh.at[ci_all.at[pl.ds(g * _W, _W)]], a_v, sc_)

        def wait_gathers(g, o_v, a_v, st, sc_):
            pltpu.make_async_copy(
                tok_hbm.at[i_all.at[pl.ds(g * _W, _W)]], o_v, st
            ).wait()
            pltpu.make_async_copy(
                comb_sh.at[ci_all.at[pl.ds(g * _W, _W)]], a_v, sc_
            ).wait()

        H = _W // 4

        def _adds_span(o_v, a_v, lo):
            @pl.loop(lo, lo + H, step=4)
            def _(r):
                for dr in range(4):
                    for c in range(0, D, 16):
                        plsc.addupdate(
                            o_v.at[r + dr, pl.ds(c, 16)],
                            a_v[r + dr, pl.ds(c, 16)],
                        )

        def adds_and_writeback(g, o_v, a_v, so):
            for q in range(4):
                _adds_span(o_v, a_v, q * H)
                pltpu.async_copy(
                    o_v.at[pl.ds(q * H, H)],
                    out_hbm.at[pl.ds(base + g * _W + q * H, H)], so)

        def wait_writeback(g, o_v, so):
            for q in range(4):
                pltpu.make_async_copy(
                    o_v.at[pl.ds(q * H, H)],
                    out_hbm.at[pl.ds(base + g * _W + q * H, H)], so
                ).wait()

        issue(0, o0, a0, st0, sc0)
        issue(1, o1, a1, st1, sc1)
        wait_gathers(0, o0, a0, st0, sc0)
        adds_and_writeback(0, o0, a0, so0)

        @pl.loop(1, nw - 1, step=2)
        def _(g2):
            wait_gathers(g2, o1, a1, st1, sc1)
            wait_writeback(g2 - 1, o0, so0)
            issue(g2 + 1, o0, a0, st0, sc0)
            adds_and_writeback(g2, o1, a1, so1)
            wait_gathers(g2 + 1, o0, a0, st0, sc0)
            wait_writeback(g2, o1, so1)
            issue(g2 + 2, o1, a1, st1, sc1)
            adds_and_writeback(g2 + 1, o0, a0, so0)

        wait_gathers(nw - 1, o1, a1, st1, sc1)
        adds_and_writeback(nw - 1, o1, a1, so1)
        wait_writeback(nw - 2, o0, so0)
        wait_writeback(nw - 1, o1, so1)

    return k


def kernel(sequence, segment_labels, token_table, segment_table, pos_table):
    B, S = sequence.shape
    V, D = token_table.shape
    C = segment_table.shape[0]
    comb = (pos_table[:, None, :] + segment_table[None, :, :]).reshape(S * C, D)
    seq_flat = sequence.reshape(-1).astype(jnp.int32)
    cidx = (
        jnp.arange(S, dtype=jnp.int32)[None, :] * C
        + segment_labels.astype(jnp.int32)
    ).reshape(-1)
    out = _build(B * S, D, S * C)(seq_flat, cidx, token_table, comb)
    return out.reshape(B, S, D)


# speedup vs baseline: 1.0065x; 1.0065x over previous
import functools

import jax
import jax.numpy as jnp
from jax import lax
from jax.experimental import pallas as pl
from jax.experimental.pallas import tpu as pltpu
from jax.experimental.pallas import tpu_sc as plsc

_W = 128
_NC, _NS = 2, 16


@functools.lru_cache(maxsize=None)
def _build(N, D, C):
    nworkers = _NC * _NS
    R = N // nworkers
    nw = R // _W
    mesh = plsc.VectorSubcoreMesh(core_axis_name="c", subcore_axis_name="s")

    @functools.partial(
        pl.kernel,
        out_type=jax.ShapeDtypeStruct((N, D), jnp.float32),
        mesh=mesh,
        scratch_types=[
            pltpu.VMEM((R,), jnp.int32),
            pltpu.VMEM((R,), jnp.int32),
            pltpu.VMEM((_W, D), jnp.float32),
            pltpu.VMEM((_W, D), jnp.float32),
            pltpu.VMEM((_W, D), jnp.float32),
            pltpu.VMEM((_W, D), jnp.float32),
            pltpu.VMEM_SHARED((C, D), jnp.float32),
            pltpu.SemaphoreType.DMA,
            pltpu.SemaphoreType.DMA,
            pltpu.SemaphoreType.DMA,
            pltpu.SemaphoreType.DMA,
            pltpu.SemaphoreType.DMA,
            pltpu.SemaphoreType.DMA,
        ],
    )
    def k(seq_hbm, cidx_hbm, tok_hbm, comb_hbm, out_hbm,
          i_all, ci_all, o0, o1, a0, a1, comb_sh,
          st0, st1, sc0, sc1, so0, so1):
        sid = lax.axis_index("s")
        wid = sid * _NC + lax.axis_index("c")
        base = wid * R

        @pl.when(sid == 0)
        def _():
            pltpu.sync_copy(comb_hbm, comb_sh)

        plsc.subcore_barrier()
        pltpu.sync_copy(seq_hbm.at[pl.ds(base, R)], i_all)
        pltpu.sync_copy(cidx_hbm.at[pl.ds(base, R)], ci_all)

        def issue(g, o_v, a_v, st, sc_):
            pltpu.async_copy(tok_hbm.at[i_all.at[pl.ds(g * _W, _W)]], o_v, st)
            pltpu.async_copy(comb_sh.at[ci_all.at[pl.ds(g * _W, _W)]], a_v, sc_)

        def wait_gathers(g, o_v, a_v, st, sc_):
            pltpu.make_async_copy(
                tok_hbm.at[i_all.at[pl.ds(g * _W, _W)]], o_v, st
            ).wait()
            pltpu.make_async_copy(
                comb_sh.at[ci_all.at[pl.ds(g * _W, _W)]], a_v, sc_
            ).wait()

        H = _W // 2

        def _adds_span(o_v, a_v, lo):
            @pl.loop(lo, lo + H, step=4)
            def _(r):
                for dr in range(4):
                    for c in range(0, D, 16):
                        plsc.addupdate(
                            o_v.at[r + dr, pl.ds(c, 16)],
                            a_v[r + dr, pl.ds(c, 16)],
                        )

        def adds_and_writeback(g, o_v, a_v, so):
            for q in range(2):
                _adds_span(o_v, a_v, q * H)
                pltpu.async_copy(
                    o_v.at[pl.ds(q * H, H)],
                    out_hbm.at[pl.ds(base + g * _W + q * H, H)], so)

        def wait_writeback(g, o_v, so):
            for q in range(2):
                pltpu.make_async_copy(
                    o_v.at[pl.ds(q * H, H)],
                    out_hbm.at[pl.ds(base + g * _W + q * H, H)], so
                ).wait()

        issue(0, o0, a0, st0, sc0)
        issue(1, o1, a1, st1, sc1)
        wait_gathers(0, o0, a0, st0, sc0)
        adds_and_writeback(0, o0, a0, so0)

        @pl.loop(1, nw - 1, step=2)
        def _(g2):
            wait_gathers(g2, o1, a1, st1, sc1)
            wait_writeback(g2 - 1, o0, so0)
            issue(g2 + 1, o0, a0, st0, sc0)
            adds_and_writeback(g2, o1, a1, so1)
            wait_gathers(g2 + 1, o0, a0, st0, sc0)
            wait_writeback(g2, o1, so1)
            issue(g2 + 2, o1, a1, st1, sc1)
            adds_and_writeback(g2 + 1, o0, a0, so0)

        wait_gathers(nw - 1, o1, a1, st1, sc1)
        adds_and_writeback(nw - 1, o1, a1, so1)
        wait_writeback(nw - 2, o0, so0)
        wait_writeback(nw - 1, o1, so1)

    return k


def kernel(sequence, segment_labels, token_table, segment_table, pos_table):
    B, S = sequence.shape
    V, D = token_table.shape
    C = segment_table.shape[0]
    comb = (pos_table[:, None, :] + segment_table[None, :, :]).reshape(S * C, D)
    seq_flat = sequence.reshape(-1).astype(jnp.int32)
    cidx = (
        jnp.arange(S, dtype=jnp.int32)[None, :] * C
        + segment_labels.astype(jnp.int32)
    ).reshape(-1)
    out = _build(B * S, D, S * C)(seq_flat, cidx, token_table, comb)
    return out.reshape(B, S, D)
